# baseline (speedup 1.0000x reference)
import functools

import jax
import jax.numpy as jnp
from jax import lax
from jax.experimental import pallas as pl
from jax.experimental.pallas import tpu as pltpu
from jax.experimental.pallas import tpu_sc as plsc

N = 10000
E = 320000
IN_DIM = 128
OUT_DIM = 64

NUM_WORKERS = 32
CHUNK = 128
CHUNKS_PER_WORKER = 80
E_PAD = NUM_WORKERS * CHUNKS_PER_WORKER * CHUNK
M_ROWS = 10240
STRIPE = M_ROWS // 16

ROW_BLOCK = 1000


def _tc_a_body(f_ref, wl_ref, wd_ref, nf_ref, z_ref):
    nf = jax.lax.dot_general(
        f_ref[...], wl_ref[...], (((1,), (1,)), ((), ())),
        preferred_element_type=jnp.float32)
    nf_ref[...] = nf
    u1 = jnp.tanh(nf)
    z_ref[...] = jax.lax.dot_general(
        u1, wd_ref[...], (((1,), (1,)), ((), ())),
        preferred_element_type=jnp.float32)


def _tc_b_body(nf_ref, s0_ref, s1_ref, out_ref):
    m = s0_ref[...] + s1_ref[...]
    out_ref[...] = jnp.tanh(nf_ref[...] + jnp.maximum(m, 0.0))


def _sc_body(z_hbm, src_hbm, dst2d_hbm, zeros_hbm,
             s0_hbm, s1_hbm,
             src_v, dst_v, rows0_v, rows1_v, z_spmem, acc_spmem, sem0, sem1):
    cid = lax.axis_index("c")
    sid = lax.axis_index("s")
    wid = sid * 2 + cid

    pltpu.sync_copy(zeros_hbm.at[pl.ds(sid * STRIPE, STRIPE)],
                    acc_spmem.at[pl.ds(sid * STRIPE, STRIPE)])
    pltpu.sync_copy(z_hbm.at[pl.ds(sid * STRIPE, STRIPE)],
                    z_spmem.at[pl.ds(sid * STRIPE, STRIPE)])

    pltpu.sync_copy(
        src_hbm.at[pl.ds(wid * CHUNKS_PER_WORKER * CHUNK,
                         CHUNKS_PER_WORKER * CHUNK)], src_v)
    pltpu.sync_copy(
        dst2d_hbm.at[pl.ds(wid * CHUNKS_PER_WORKER, CHUNKS_PER_WORKER)], dst_v)

    plsc.subcore_barrier()

    bufs = ((rows0_v, sem0), (rows1_v, sem1))

    def gather(c, buf, sem):
        return pltpu.async_copy(
            z_spmem.at[src_v.at[pl.ds(c * CHUNK, CHUNK)]], buf, sem)

    gather(0, rows0_v, sem0)
    gather(1, rows1_v, sem1)

    def outer(g, carry):
        for b in range(2):
            c = g * 2 + b
            buf, sem = bufs[b]
            pltpu.make_async_copy(
                z_spmem.at[src_v.at[pl.ds(c * CHUNK, CHUNK)]], buf, sem).wait()
            pltpu.sync_copy(buf, acc_spmem.at[dst_v.at[c]], add=True)

            @pl.when(c + 2 < CHUNKS_PER_WORKER)
            def _():
                gather(c + 2, buf, sem)
        return carry

    lax.fori_loop(0, CHUNKS_PER_WORKER // 2, outer, 0, unroll=False)

    plsc.subcore_barrier()

    @pl.when(cid == 0)
    def _():
        pltpu.sync_copy(acc_spmem.at[pl.ds(sid * STRIPE, STRIPE)],
                        s0_hbm.at[pl.ds(sid * STRIPE, STRIPE)])

    @pl.when(cid == 1)
    def _():
        pltpu.sync_copy(acc_spmem.at[pl.ds(sid * STRIPE, STRIPE)],
                        s1_hbm.at[pl.ds(sid * STRIPE, STRIPE)])


@jax.jit
def kernel(feature, edge_index, W_lin, W_dense):
    n_blocks = N // ROW_BLOCK

    nf, z = pl.pallas_call(
        _tc_a_body,
        grid=(n_blocks,),
        in_specs=[
            pl.BlockSpec((ROW_BLOCK, IN_DIM), lambda r: (r, 0)),
            pl.BlockSpec((OUT_DIM, IN_DIM), lambda r: (0, 0)),
            pl.BlockSpec((OUT_DIM, OUT_DIM), lambda r: (0, 0)),
        ],
        out_specs=[
            pl.BlockSpec((ROW_BLOCK, OUT_DIM), lambda r: (r, 0)),
            pl.BlockSpec((ROW_BLOCK, OUT_DIM), lambda r: (r, 0)),
        ],
        out_shape=[
            jax.ShapeDtypeStruct((N, OUT_DIM), jnp.float32),
            jax.ShapeDtypeStruct((M_ROWS, OUT_DIM), jnp.float32),
        ],
    )(feature, W_lin, W_dense)

    src = jnp.concatenate(
        [edge_index[0], jnp.zeros((E_PAD - E,), jnp.int32)])
    dst = jnp.concatenate(
        [edge_index[1], jnp.full((E_PAD - E,), N, jnp.int32)])
    dst2d = dst.reshape(E_PAD // CHUNK, CHUNK)
    zeros_hbm = jnp.zeros((M_ROWS, OUT_DIM), jnp.float32)

    sc_fn = pl.kernel(
        _sc_body,
        out_type=[
            jax.ShapeDtypeStruct((M_ROWS, OUT_DIM), jnp.float32),
            jax.ShapeDtypeStruct((M_ROWS, OUT_DIM), jnp.float32),
        ],
        mesh=plsc.VectorSubcoreMesh(core_axis_name="c", subcore_axis_name="s"),
        compiler_params=pltpu.CompilerParams(use_tc_tiling_on_sc=False),
        scratch_types=[
            pltpu.VMEM((CHUNKS_PER_WORKER * CHUNK,), jnp.int32),
            pltpu.VMEM((CHUNKS_PER_WORKER, CHUNK), jnp.int32),
            pltpu.VMEM((CHUNK, OUT_DIM), jnp.float32),
            pltpu.VMEM((CHUNK, OUT_DIM), jnp.float32),
            pltpu.VMEM_SHARED((M_ROWS, OUT_DIM), jnp.float32),
            pltpu.VMEM_SHARED((M_ROWS, OUT_DIM), jnp.float32),
            pltpu.SemaphoreType.DMA,
            pltpu.SemaphoreType.DMA,
        ],
    )
    s0, s1 = sc_fn(z, src, dst2d, zeros_hbm)

    out = pl.pallas_call(
        _tc_b_body,
        grid=(n_blocks,),
        in_specs=[
            pl.BlockSpec((ROW_BLOCK, OUT_DIM), lambda r: (r, 0)),
            pl.BlockSpec((ROW_BLOCK, OUT_DIM), lambda r: (r, 0)),
            pl.BlockSpec((ROW_BLOCK, OUT_DIM), lambda r: (r, 0)),
        ],
        out_specs=pl.BlockSpec((ROW_BLOCK, OUT_DIM), lambda r: (r, 0)),
        out_shape=jax.ShapeDtypeStruct((N, OUT_DIM), jnp.float32),
    )(nf, s0, s1)
    return out[:N]


# speedup vs baseline: 1.0091x; 1.0091x over previous
import functools

import jax
import jax.numpy as jnp
from jax import lax
from jax.experimental import pallas as pl
from jax.experimental.pallas import tpu as pltpu
from jax.experimental.pallas import tpu_sc as plsc

N = 10000
E = 320000
IN_DIM = 128
OUT_DIM = 64

NUM_WORKERS = 32
CHUNK = 128
CHUNKS_PER_WORKER = 80
E_PAD = NUM_WORKERS * CHUNKS_PER_WORKER * CHUNK
M_ROWS = 10240
STRIPE = M_ROWS // 16

ROW_BLOCK = 1000


def _tc_a_body(f_ref, wl_ref, wd_ref, nf_ref, z_ref):
    nf = jax.lax.dot_general(
        f_ref[...], wl_ref[...], (((1,), (1,)), ((), ())),
        preferred_element_type=jnp.float32)
    nf_ref[...] = nf
    u1 = jnp.tanh(nf)
    z_ref[...] = jax.lax.dot_general(
        u1, wd_ref[...], (((1,), (1,)), ((), ())),
        preferred_element_type=jnp.float32)


def _tc_b_body(nf_ref, s0_ref, s1_ref, out_ref):
    m = s0_ref[...] + s1_ref[...]
    out_ref[...] = jnp.tanh(nf_ref[...] + jnp.maximum(m, 0.0))


NBUF = 3
NFULL = CHUNKS_PER_WORKER // NBUF
TAIL = CHUNKS_PER_WORKER - NFULL * NBUF


def _sc_body(z_hbm, src_hbm, dst2d_hbm, zeros_hbm,
             s0_hbm, s1_hbm,
             src_v, dst_v, rb0, rb1, rb2, z_spmem, acc_spmem,
             sm0, sm1, sm2):
    rows_bufs = (rb0, rb1, rb2)
    sems = (sm0, sm1, sm2)
    cid = lax.axis_index("c")
    sid = lax.axis_index("s")
    wid = sid * 2 + cid

    pltpu.sync_copy(zeros_hbm.at[pl.ds(sid * STRIPE, STRIPE)],
                    acc_spmem.at[pl.ds(sid * STRIPE, STRIPE)])
    pltpu.sync_copy(z_hbm.at[pl.ds(sid * STRIPE, STRIPE)],
                    z_spmem.at[pl.ds(sid * STRIPE, STRIPE)])

    pltpu.sync_copy(
        src_hbm.at[pl.ds(wid * CHUNKS_PER_WORKER * CHUNK,
                         CHUNKS_PER_WORKER * CHUNK)], src_v)
    pltpu.sync_copy(
        dst2d_hbm.at[pl.ds(wid * CHUNKS_PER_WORKER, CHUNKS_PER_WORKER)], dst_v)

    plsc.subcore_barrier()

    def gather(c, buf, sem):
        return pltpu.async_copy(
            z_spmem.at[src_v.at[pl.ds(c * CHUNK, CHUNK)]], buf, sem)

    for b in range(NBUF):
        gather(b, rows_bufs[b], sems[b])

    def outer(g, carry):
        for b in range(NBUF):
            c = g * NBUF + b
            buf, sem = rows_bufs[b], sems[b]
            pltpu.make_async_copy(
                z_spmem.at[src_v.at[pl.ds(c * CHUNK, CHUNK)]], buf, sem).wait()
            pltpu.sync_copy(buf, acc_spmem.at[dst_v.at[c]], add=True)

            @pl.when(c + NBUF < CHUNKS_PER_WORKER)
            def _():
                gather(c + NBUF, buf, sem)
        return carry

    lax.fori_loop(0, NFULL, outer, 0, unroll=False)

    for t in range(TAIL):
        c = NFULL * NBUF + t
        b = c % NBUF
        buf, sem = rows_bufs[b], sems[b]
        pltpu.make_async_copy(
            z_spmem.at[src_v.at[pl.ds(c * CHUNK, CHUNK)]], buf, sem).wait()
        pltpu.sync_copy(buf, acc_spmem.at[dst_v.at[c]], add=True)

    plsc.subcore_barrier()

    @pl.when(cid == 0)
    def _():
        pltpu.sync_copy(acc_spmem.at[pl.ds(sid * STRIPE, STRIPE)],
                        s0_hbm.at[pl.ds(sid * STRIPE, STRIPE)])

    @pl.when(cid == 1)
    def _():
        pltpu.sync_copy(acc_spmem.at[pl.ds(sid * STRIPE, STRIPE)],
                        s1_hbm.at[pl.ds(sid * STRIPE, STRIPE)])


@jax.jit
def kernel(feature, edge_index, W_lin, W_dense):
    n_blocks = N // ROW_BLOCK

    nf, z = pl.pallas_call(
        _tc_a_body,
        grid=(n_blocks,),
        in_specs=[
            pl.BlockSpec((ROW_BLOCK, IN_DIM), lambda r: (r, 0)),
            pl.BlockSpec((OUT_DIM, IN_DIM), lambda r: (0, 0)),
            pl.BlockSpec((OUT_DIM, OUT_DIM), lambda r: (0, 0)),
        ],
        out_specs=[
            pl.BlockSpec((ROW_BLOCK, OUT_DIM), lambda r: (r, 0)),
            pl.BlockSpec((ROW_BLOCK, OUT_DIM), lambda r: (r, 0)),
        ],
        out_shape=[
            jax.ShapeDtypeStruct((N, OUT_DIM), jnp.float32),
            jax.ShapeDtypeStruct((M_ROWS, OUT_DIM), jnp.float32),
        ],
    )(feature, W_lin, W_dense)

    src = jnp.concatenate(
        [edge_index[0], jnp.zeros((E_PAD - E,), jnp.int32)])
    dst = jnp.concatenate(
        [edge_index[1], jnp.full((E_PAD - E,), N, jnp.int32)])
    dst2d = dst.reshape(E_PAD // CHUNK, CHUNK)
    zeros_hbm = jnp.zeros((M_ROWS, OUT_DIM), jnp.float32)

    sc_fn = pl.kernel(
        _sc_body,
        out_type=[
            jax.ShapeDtypeStruct((M_ROWS, OUT_DIM), jnp.float32),
            jax.ShapeDtypeStruct((M_ROWS, OUT_DIM), jnp.float32),
        ],
        mesh=plsc.VectorSubcoreMesh(core_axis_name="c", subcore_axis_name="s"),
        compiler_params=pltpu.CompilerParams(use_tc_tiling_on_sc=False),
        scratch_types=[
            pltpu.VMEM((CHUNKS_PER_WORKER * CHUNK,), jnp.int32),
            pltpu.VMEM((CHUNKS_PER_WORKER, CHUNK), jnp.int32),
            pltpu.VMEM((CHUNK, OUT_DIM), jnp.float32),
            pltpu.VMEM((CHUNK, OUT_DIM), jnp.float32),
            pltpu.VMEM((CHUNK, OUT_DIM), jnp.float32),
            pltpu.VMEM_SHARED((M_ROWS, OUT_DIM), jnp.float32),
            pltpu.VMEM_SHARED((M_ROWS, OUT_DIM), jnp.float32),
            pltpu.SemaphoreType.DMA,
            pltpu.SemaphoreType.DMA,
            pltpu.SemaphoreType.DMA,
        ],
    )
    s0, s1 = sc_fn(z, src, dst2d, zeros_hbm)

    out = pl.pallas_call(
        _tc_b_body,
        grid=(n_blocks,),
        in_specs=[
            pl.BlockSpec((ROW_BLOCK, OUT_DIM), lambda r: (r, 0)),
            pl.BlockSpec((ROW_BLOCK, OUT_DIM), lambda r: (r, 0)),
            pl.BlockSpec((ROW_BLOCK, OUT_DIM), lambda r: (r, 0)),
        ],
        out_specs=pl.BlockSpec((ROW_BLOCK, OUT_DIM), lambda r: (r, 0)),
        out_shape=jax.ShapeDtypeStruct((N, OUT_DIM), jnp.float32),
    )(nf, s0, s1)
    return out[:N]


# speedup vs baseline: 1.1239x; 1.1138x over previous
import functools

import jax
import jax.numpy as jnp
from jax import lax
from jax.experimental import pallas as pl
from jax.experimental.pallas import tpu as pltpu
from jax.experimental.pallas import tpu_sc as plsc

N = 10000
E = 320000
IN_DIM = 128
OUT_DIM = 64

NUM_WORKERS = 32
CHUNK = 128
CHUNKS_PER_WORKER = 80
REAL_CHUNKS = E // CHUNK
E_PAD = NUM_WORKERS * CHUNKS_PER_WORKER * CHUNK
M_ROWS = 10240
STRIPE = M_ROWS // 16
ZROWS = 64

ROW_BLOCK = 1000
E_BLOCK = E_PAD // 10

NBUF = 3
NFULL = CHUNKS_PER_WORKER // NBUF
TAIL = CHUNKS_PER_WORKER - NFULL * NBUF


def _tc_a_body(f_ref, ei_ref, wl_ref, wd_ref, nf_ref, z_ref, src_ref, dst_ref):
    nf = jax.lax.dot_general(
        f_ref[...], wl_ref[...], (((1,), (1,)), ((), ())),
        preferred_element_type=jnp.float32)
    nf_ref[...] = nf
    u1 = jnp.tanh(nf)
    z_ref[...] = jax.lax.dot_general(
        u1, wd_ref[...], (((1,), (1,)), ((), ())),
        preferred_element_type=jnp.float32)
    src_ref[...] = ei_ref[0, :]
    dst_ref[...] = ei_ref[1, :]


def _tc_b_body(nf_ref, s0_ref, s1_ref, out_ref):
    m = s0_ref[...] + s1_ref[...]
    out_ref[...] = jnp.tanh(nf_ref[...] + jnp.maximum(m, 0.0))


def _sc_body(z_hbm, src_hbm, dst_hbm,
             s0_hbm, s1_hbm,
             src_v, zb_v, rb0, rb1, rb2, db0, db1, db2,
             z_spmem, acc_spmem,
             gs0, gs1, gs2, ds0, ds1, ds2):
    rows_bufs = (rb0, rb1, rb2)
    dst_bufs = (db0, db1, db2)
    gsems = (gs0, gs1, gs2)
    dsems = (ds0, ds1, ds2)

    cid = lax.axis_index("c")
    sid = lax.axis_index("s")
    wid = sid * 2 + cid
    base_chunk = wid * CHUNKS_PER_WORKER

    def zrow(i, carry):
        for j in range(OUT_DIM // 16):
            zb_v[i, pl.ds(j * 16, 16)] = jnp.zeros((16,), jnp.float32)
        return carry

    lax.fori_loop(0, ZROWS, zrow, 0, unroll=False)
    for k in range(STRIPE // ZROWS):
        pltpu.sync_copy(zb_v,
                        acc_spmem.at[pl.ds(sid * STRIPE + k * ZROWS, ZROWS)])

    pltpu.sync_copy(
        src_hbm.at[pl.ds(base_chunk * CHUNK, CHUNKS_PER_WORKER * CHUNK)],
        src_v)
    pltpu.sync_copy(z_hbm.at[pl.ds(sid * STRIPE, STRIPE)],
                    z_spmem.at[pl.ds(sid * STRIPE, STRIPE)])

    plsc.subcore_barrier()

    def issue(c, b):
        @pl.when((c < CHUNKS_PER_WORKER) & (base_chunk + c < REAL_CHUNKS))
        def _():
            pltpu.async_copy(
                dst_hbm.at[pl.ds((base_chunk + c) * CHUNK, CHUNK)],
                dst_bufs[b], dsems[b])
            pltpu.async_copy(
                z_spmem.at[src_v.at[pl.ds(c * CHUNK, CHUNK)]],
                rows_bufs[b], gsems[b])

    def drain(c, b):
        @pl.when(base_chunk + c < REAL_CHUNKS)
        def _():
            pltpu.make_async_copy(
                dst_hbm.at[pl.ds((base_chunk + c) * CHUNK, CHUNK)],
                dst_bufs[b], dsems[b]).wait()
            pltpu.make_async_copy(
                z_spmem.at[src_v.at[pl.ds(c * CHUNK, CHUNK)]],
                rows_bufs[b], gsems[b]).wait()
            pltpu.sync_copy(rows_bufs[b], acc_spmem.at[dst_bufs[b]], add=True)

    for b in range(NBUF):
        issue(b, b)

    def outer(g, carry):
        for b in range(NBUF):
            c = g * NBUF + b
            drain(c, b)
            issue(c + NBUF, b)
        return carry

    lax.fori_loop(0, NFULL, outer, 0, unroll=False)

    for t in range(TAIL):
        c = NFULL * NBUF + t
        drain(c, c % NBUF)

    plsc.subcore_barrier()

    @pl.when(cid == 0)
    def _():
        pltpu.sync_copy(acc_spmem.at[pl.ds(sid * STRIPE, STRIPE)],
                        s0_hbm.at[pl.ds(sid * STRIPE, STRIPE)])

    @pl.when(cid == 1)
    def _():
        pltpu.sync_copy(acc_spmem.at[pl.ds(sid * STRIPE, STRIPE)],
                        s1_hbm.at[pl.ds(sid * STRIPE, STRIPE)])


@jax.jit
def kernel(feature, edge_index, W_lin, W_dense):
    n_blocks = N // ROW_BLOCK

    nf, z, src, dst = pl.pallas_call(
        _tc_a_body,
        grid=(n_blocks,),
        in_specs=[
            pl.BlockSpec((ROW_BLOCK, IN_DIM), lambda r: (r, 0)),
            pl.BlockSpec((2, E_BLOCK), lambda r: (0, r)),
            pl.BlockSpec((OUT_DIM, IN_DIM), lambda r: (0, 0)),
            pl.BlockSpec((OUT_DIM, OUT_DIM), lambda r: (0, 0)),
        ],
        out_specs=[
            pl.BlockSpec((ROW_BLOCK, OUT_DIM), lambda r: (r, 0)),
            pl.BlockSpec((ROW_BLOCK, OUT_DIM), lambda r: (r, 0)),
            pl.BlockSpec((E_BLOCK,), lambda r: (r,)),
            pl.BlockSpec((E_BLOCK,), lambda r: (r,)),
        ],
        out_shape=[
            jax.ShapeDtypeStruct((N, OUT_DIM), jnp.float32),
            jax.ShapeDtypeStruct((M_ROWS, OUT_DIM), jnp.float32),
            jax.ShapeDtypeStruct((E_PAD,), jnp.int32),
            jax.ShapeDtypeStruct((E_PAD,), jnp.int32),
        ],
    )(feature, edge_index, W_lin, W_dense)

    sc_fn = pl.kernel(
        _sc_body,
        out_type=[
            jax.ShapeDtypeStruct((M_ROWS, OUT_DIM), jnp.float32),
            jax.ShapeDtypeStruct((M_ROWS, OUT_DIM), jnp.float32),
        ],
        mesh=plsc.VectorSubcoreMesh(core_axis_name="c", subcore_axis_name="s"),
        compiler_params=pltpu.CompilerParams(use_tc_tiling_on_sc=False),
        scratch_types=[
            pltpu.VMEM((CHUNKS_PER_WORKER * CHUNK,), jnp.int32),
            pltpu.VMEM((ZROWS, OUT_DIM), jnp.float32),
            pltpu.VMEM((CHUNK, OUT_DIM), jnp.float32),
            pltpu.VMEM((CHUNK, OUT_DIM), jnp.float32),
            pltpu.VMEM((CHUNK, OUT_DIM), jnp.float32),
            pltpu.VMEM((CHUNK,), jnp.int32),
            pltpu.VMEM((CHUNK,), jnp.int32),
            pltpu.VMEM((CHUNK,), jnp.int32),
            pltpu.VMEM_SHARED((M_ROWS, OUT_DIM), jnp.float32),
            pltpu.VMEM_SHARED((M_ROWS, OUT_DIM), jnp.float32),
            pltpu.SemaphoreType.DMA,
            pltpu.SemaphoreType.DMA,
            pltpu.SemaphoreType.DMA,
            pltpu.SemaphoreType.DMA,
            pltpu.SemaphoreType.DMA,
            pltpu.SemaphoreType.DMA,
        ],
    )
    s0, s1 = sc_fn(z, src, dst)

    out = pl.pallas_call(
        _tc_b_body,
        grid=(n_blocks,),
        in_specs=[
            pl.BlockSpec((ROW_BLOCK, OUT_DIM), lambda r: (r, 0)),
            pl.BlockSpec((ROW_BLOCK, OUT_DIM), lambda r: (r, 0)),
            pl.BlockSpec((ROW_BLOCK, OUT_DIM), lambda r: (r, 0)),
        ],
        out_specs=pl.BlockSpec((ROW_BLOCK, OUT_DIM), lambda r: (r, 0)),
        out_shape=jax.ShapeDtypeStruct((N, OUT_DIM), jnp.float32),
    )(nf, s0, s1)
    return out


# speedup vs baseline: 1.1455x; 1.0192x over previous
import functools

import jax
import jax.numpy as jnp
from jax import lax
from jax.experimental import pallas as pl
from jax.experimental.pallas import tpu as pltpu
from jax.experimental.pallas import tpu_sc as plsc

N = 10000
E = 320000
IN_DIM = 128
OUT_DIM = 64

NUM_WORKERS = 32
CHUNK = 128
CHUNKS_PER_WORKER = 80
REAL_CHUNKS = E // CHUNK
E_PAD = NUM_WORKERS * CHUNKS_PER_WORKER * CHUNK
M_ROWS = 10240
STRIPE = M_ROWS // 16
ZROWS = 64

ROW_BLOCK = 1000
E_BLOCK = E_PAD // 10

NBUF = 3
NFULL = CHUNKS_PER_WORKER // NBUF
TAIL = CHUNKS_PER_WORKER - NFULL * NBUF


def _tc_a_body(f_ref, ei_ref, wl_ref, wd_ref, nf_ref, z_ref, src_ref, dst_ref):
    nf = jax.lax.dot_general(
        f_ref[...], wl_ref[...], (((1,), (1,)), ((), ())),
        preferred_element_type=jnp.float32)
    nf_ref[...] = nf
    u1 = jnp.tanh(nf)
    z_ref[...] = jax.lax.dot_general(
        u1, wd_ref[...], (((1,), (1,)), ((), ())),
        preferred_element_type=jnp.float32)
    src_ref[...] = ei_ref[0, :]
    dst_ref[...] = ei_ref[1, :]


def _tc_b_body(nf_ref, s0_ref, s1_ref, out_ref):
    m = s0_ref[...] + s1_ref[...]
    out_ref[...] = jnp.tanh(nf_ref[...] + jnp.maximum(m, 0.0))


def _sc_body(z_hbm, src_hbm, dst_hbm,
             s0_hbm, s1_hbm,
             src_v, zb_v, rb0, rb1, rb2, db0, db1, db2,
             z_spmem, acc_spmem,
             gs0, gs1, gs2, ds0, ds1, ds2):
    rows_bufs = (rb0, rb1, rb2)
    dst_bufs = (db0, db1, db2)
    gsems = (gs0, gs1, gs2)
    dsems = (ds0, ds1, ds2)

    cid = lax.axis_index("c")
    sid = lax.axis_index("s")
    wid = sid * 2 + cid
    base_chunk = wid * CHUNKS_PER_WORKER

    def zrow(i, carry):
        for j in range(OUT_DIM // 16):
            zb_v[i, pl.ds(j * 16, 16)] = jnp.zeros((16,), jnp.float32)
        return carry

    lax.fori_loop(0, ZROWS, zrow, 0, unroll=False)

    zero_copies = [
        pltpu.async_copy(
            zb_v, acc_spmem.at[pl.ds(sid * STRIPE + k * ZROWS, ZROWS)], gs0)
        for k in range(STRIPE // ZROWS)]
    src_copy = pltpu.async_copy(
        src_hbm.at[pl.ds(base_chunk * CHUNK, CHUNKS_PER_WORKER * CHUNK)],
        src_v, gs1)
    z_copy = pltpu.async_copy(
        z_hbm.at[pl.ds(sid * STRIPE, STRIPE)],
        z_spmem.at[pl.ds(sid * STRIPE, STRIPE)], gs2)
    for cp in zero_copies:
        cp.wait()
    src_copy.wait()
    z_copy.wait()

    plsc.subcore_barrier()

    def issue(c, b):
        @pl.when((c < CHUNKS_PER_WORKER) & (base_chunk + c < REAL_CHUNKS))
        def _():
            pltpu.async_copy(
                dst_hbm.at[pl.ds((base_chunk + c) * CHUNK, CHUNK)],
                dst_bufs[b], dsems[b])
            pltpu.async_copy(
                z_spmem.at[src_v.at[pl.ds(c * CHUNK, CHUNK)]],
                rows_bufs[b], gsems[b])

    def drain(c, b):
        @pl.when(base_chunk + c < REAL_CHUNKS)
        def _():
            pltpu.make_async_copy(
                dst_hbm.at[pl.ds((base_chunk + c) * CHUNK, CHUNK)],
                dst_bufs[b], dsems[b]).wait()
            pltpu.make_async_copy(
                z_spmem.at[src_v.at[pl.ds(c * CHUNK, CHUNK)]],
                rows_bufs[b], gsems[b]).wait()
            pltpu.sync_copy(rows_bufs[b], acc_spmem.at[dst_bufs[b]], add=True)

    for b in range(NBUF):
        issue(b, b)

    def outer(g, carry):
        for b in range(NBUF):
            c = g * NBUF + b
            drain(c, b)
            issue(c + NBUF, b)
        return carry

    lax.fori_loop(0, NFULL, outer, 0, unroll=False)

    for t in range(TAIL):
        c = NFULL * NBUF + t
        drain(c, c % NBUF)

    plsc.subcore_barrier()

    @pl.when(cid == 0)
    def _():
        pltpu.sync_copy(acc_spmem.at[pl.ds(sid * STRIPE, STRIPE)],
                        s0_hbm.at[pl.ds(sid * STRIPE, STRIPE)])

    @pl.when(cid == 1)
    def _():
        pltpu.sync_copy(acc_spmem.at[pl.ds(sid * STRIPE, STRIPE)],
                        s1_hbm.at[pl.ds(sid * STRIPE, STRIPE)])


@jax.jit
def kernel(feature, edge_index, W_lin, W_dense):
    n_blocks = N // ROW_BLOCK

    nf, z, src, dst = pl.pallas_call(
        _tc_a_body,
        grid=(n_blocks,),
        in_specs=[
            pl.BlockSpec((ROW_BLOCK, IN_DIM), lambda r: (r, 0)),
            pl.BlockSpec((2, E_BLOCK), lambda r: (0, r)),
            pl.BlockSpec((OUT_DIM, IN_DIM), lambda r: (0, 0)),
            pl.BlockSpec((OUT_DIM, OUT_DIM), lambda r: (0, 0)),
        ],
        out_specs=[
            pl.BlockSpec((ROW_BLOCK, OUT_DIM), lambda r: (r, 0)),
            pl.BlockSpec((ROW_BLOCK, OUT_DIM), lambda r: (r, 0)),
            pl.BlockSpec((E_BLOCK,), lambda r: (r,)),
            pl.BlockSpec((E_BLOCK,), lambda r: (r,)),
        ],
        out_shape=[
            jax.ShapeDtypeStruct((N, OUT_DIM), jnp.float32),
            jax.ShapeDtypeStruct((M_ROWS, OUT_DIM), jnp.float32),
            jax.ShapeDtypeStruct((E_PAD,), jnp.int32),
            jax.ShapeDtypeStruct((E_PAD,), jnp.int32),
        ],
    )(feature, edge_index, W_lin, W_dense)

    sc_fn = pl.kernel(
        _sc_body,
        out_type=[
            jax.ShapeDtypeStruct((M_ROWS, OUT_DIM), jnp.float32),
            jax.ShapeDtypeStruct((M_ROWS, OUT_DIM), jnp.float32),
        ],
        mesh=plsc.VectorSubcoreMesh(core_axis_name="c", subcore_axis_name="s"),
        compiler_params=pltpu.CompilerParams(use_tc_tiling_on_sc=False),
        scratch_types=[
            pltpu.VMEM((CHUNKS_PER_WORKER * CHUNK,), jnp.int32),
            pltpu.VMEM((ZROWS, OUT_DIM), jnp.float32),
            pltpu.VMEM((CHUNK, OUT_DIM), jnp.float32),
            pltpu.VMEM((CHUNK, OUT_DIM), jnp.float32),
            pltpu.VMEM((CHUNK, OUT_DIM), jnp.float32),
            pltpu.VMEM((CHUNK,), jnp.int32),
            pltpu.VMEM((CHUNK,), jnp.int32),
            pltpu.VMEM((CHUNK,), jnp.int32),
            pltpu.VMEM_SHARED((M_ROWS, OUT_DIM), jnp.float32),
            pltpu.VMEM_SHARED((M_ROWS, OUT_DIM), jnp.float32),
            pltpu.SemaphoreType.DMA,
            pltpu.SemaphoreType.DMA,
            pltpu.SemaphoreType.DMA,
            pltpu.SemaphoreType.DMA,
            pltpu.SemaphoreType.DMA,
            pltpu.SemaphoreType.DMA,
        ],
    )
    s0, s1 = sc_fn(z, src, dst)

    out = pl.pallas_call(
        _tc_b_body,
        grid=(n_blocks,),
        in_specs=[
            pl.BlockSpec((ROW_BLOCK, OUT_DIM), lambda r: (r, 0)),
            pl.BlockSpec((ROW_BLOCK, OUT_DIM), lambda r: (r, 0)),
            pl.BlockSpec((ROW_BLOCK, OUT_DIM), lambda r: (r, 0)),
        ],
        out_specs=pl.BlockSpec((ROW_BLOCK, OUT_DIM), lambda r: (r, 0)),
        out_shape=jax.ShapeDtypeStruct((N, OUT_DIM), jnp.float32),
    )(nf, s0, s1)
    return out
